# baseline (speedup 1.0000x reference)
import functools

import jax
import jax.numpy as jnp
import numpy as np
from jax.experimental import pallas as pl
from jax.experimental.pallas import tpu as pltpu

B, T = 4, 2048
D_IN = 640
H = 1024
N_CLOCKS = 16
SHARP = 4.0
T_REL = 1.0

BM = 512


def _fused_kernel(feat_ref, w1_ref, thw_ref, wph_ref, b1na_ref, b1nm_ref,
                  naW2_ref, nmW2_ref, b2na_ref, b2nm_ref, scal_ref, out_ref):
    feat = feat_ref[...]
    na_thr = scal_ref[0, 0]
    nm_thr = scal_ref[0, 1]
    sens = scal_ref[0, 2]
    freq = scal_ref[0, 3]
    th_b = scal_ref[0, 4]

    g = jnp.dot(feat, w1_ref[...], preferred_element_type=jnp.float32)

    strength = jnp.sqrt(jnp.sum(feat * feat, axis=1, keepdims=True))
    th = jnp.sum(feat * thw_ref[...], axis=1, keepdims=True) + th_b
    theta_mod = jax.nn.sigmoid(th)

    pid = pl.program_id(0)
    t0 = jax.lax.rem(pid * BM, T).astype(jnp.float32)
    row = jax.lax.broadcasted_iota(jnp.int32, (BM, 1), 0).astype(jnp.float32)
    t_val = (t0 + row) * (T_REL / (T - 1))
    theta_phase = jnp.clip(2.0 * np.pi * freq * t_val * theta_mod,
                           -100.0, 100.0)
    nmda_window = jax.nn.sigmoid(jnp.sin(theta_phase) * SHARP)

    h1 = jnp.tanh(g[:, :H] + b1na_ref[...])
    h2 = jnp.tanh(g[:, H:] + theta_phase * wph_ref[...] + b1nm_ref[...])

    na_logits = jnp.clip(
        jnp.dot(h1, naW2_ref[...], preferred_element_type=jnp.float32)
        + b2na_ref[...], -10.0, 10.0)
    nm_logits = jnp.clip(
        jnp.dot(h2, nmW2_ref[...], preferred_element_type=jnp.float32)
        + b2nm_ref[...], -10.0, 10.0)

    thr_na = na_thr - sens * strength
    thr_nm = nm_thr - sens * strength
    act = (na_logits > thr_na) & (nm_logits >= thr_nm)
    out = jnp.where(
        act,
        jax.nn.sigmoid(na_logits) * jax.nn.sigmoid(nm_logits) * nmda_window,
        0.0)
    out_ref[...] = out


@functools.partial(jax.jit, static_argnames=("interpret",))
def _run(feat, w1cat, thw, wph, b1na, b1nm, naW2, nmW2, b2na, b2nm, scal,
         interpret=False):
    M = B * T
    feat2 = feat.reshape(M, D_IN)
    grid = (M // BM,)
    full = lambda i: (0, 0)
    out = pl.pallas_call(
        _fused_kernel,
        grid=grid,
        in_specs=[
            pl.BlockSpec((BM, D_IN), lambda i: (i, 0)),
            pl.BlockSpec((D_IN, 2 * H), full),
            pl.BlockSpec((1, D_IN), full),
            pl.BlockSpec((1, H), full),
            pl.BlockSpec((1, H), full),
            pl.BlockSpec((1, H), full),
            pl.BlockSpec((H, N_CLOCKS), full),
            pl.BlockSpec((H, N_CLOCKS), full),
            pl.BlockSpec((1, N_CLOCKS), full),
            pl.BlockSpec((1, N_CLOCKS), full),
            pl.BlockSpec((1, 8), full),
        ],
        out_specs=pl.BlockSpec((BM, N_CLOCKS), lambda i: (i, 0)),
        out_shape=jax.ShapeDtypeStruct((M, N_CLOCKS), jnp.float32),
        compiler_params=pltpu.CompilerParams(
            dimension_semantics=("arbitrary",)),
        interpret=interpret,
    )(feat2, w1cat, thw, wph, b1na, b1nm, naW2, nmW2, b2na, b2nm, scal)
    return out.reshape(B, T, N_CLOCKS)


def kernel(feat, na_W1, na_b1, na_W2, na_b2, th_W, th_b, nm_W1, nm_b1,
           nm_W2, nm_b2, na_thr, nmda_thr, sens, gate_temp, theta_freq,
           interpret=False):
    safe_sens = jnp.clip(sens, 0.0, 1.0)
    safe_freq = jnp.clip(theta_freq, 0.1, 16.0)
    w1cat = jnp.concatenate([na_W1, nm_W1[:D_IN]], axis=1)
    wph = nm_W1[D_IN].reshape(1, H)
    scal = jnp.zeros((1, 8), jnp.float32)
    scal = scal.at[0, 0].set(na_thr[0])
    scal = scal.at[0, 1].set(nmda_thr[0])
    scal = scal.at[0, 2].set(safe_sens[0])
    scal = scal.at[0, 3].set(safe_freq[0])
    scal = scal.at[0, 4].set(th_b[0])
    return _run(feat, w1cat, th_W.reshape(1, D_IN), wph,
                na_b1.reshape(1, H), nm_b1.reshape(1, H),
                na_W2, nm_W2, na_b2.reshape(1, N_CLOCKS),
                nm_b2.reshape(1, N_CLOCKS), scal, interpret=interpret)
